# baseline (speedup 1.0000x reference)
import functools

import jax
import jax.numpy as jnp
from jax import lax
from jax.experimental import pallas as pl
from jax.experimental.pallas import tpu as pltpu
from jax.experimental.pallas import tpu_sc as plsc

D = 128
NC, NS = 2, 16
NW = NC * NS
CH = 128
NBUF = 4


@functools.lru_cache(maxsize=None)
def _make_gather(batch: int, hist: int):
    BPT = batch // NW
    NCB = BPT // CH
    NCHUNK = hist * NCB
    assert NCHUNK % NBUF == 0
    mesh = plsc.VectorSubcoreMesh(core_axis_name="c", subcore_axis_name="s")

    @functools.partial(
        pl.kernel,
        out_type=jax.ShapeDtypeStruct((hist, batch, D), jnp.float32),
        mesh=mesh,
        scratch_types=[
            pltpu.VMEM((hist, BPT), jnp.int32),
            pltpu.VMEM((NBUF, CH, D), jnp.float32),
            pltpu.SemaphoreType.DMA((NBUF,)),
            pltpu.SemaphoreType.DMA((NBUF,)),
        ],
    )
    def gather_kernel(idx_hbm, table_hbm, out_hbm, idx_v, rows_v, gsem, wsem):
        wid = lax.axis_index("s") * NC + lax.axis_index("c")
        base = wid * BPT
        pltpu.sync_copy(idx_hbm.at[:, pl.ds(base, BPT)], idx_v)

        def start_gather(k, b):
            h, c = k // NCB, k % NCB
            pltpu.async_copy(
                table_hbm.at[idx_v.at[h].at[pl.ds(c * CH, CH)]],
                rows_v.at[b], gsem.at[b])

        def wait_gather(b):
            pltpu.make_async_copy(table_hbm.at[idx_v.at[0].at[pl.ds(0, CH)]],
                                  rows_v.at[b], gsem.at[b]).wait()

        def start_write(k, b):
            h, c = k // NCB, k % NCB
            pltpu.async_copy(rows_v.at[b],
                             out_hbm.at[h].at[pl.ds(base + c * CH, CH)],
                             wsem.at[b])

        def wait_write(b):
            pltpu.make_async_copy(rows_v.at[b], out_hbm.at[0].at[pl.ds(0, CH)],
                                  wsem.at[b]).wait()

        for b in range(NBUF):
            start_gather(b, b)

        @pl.loop(0, NCHUNK // NBUF)
        def _group(g):
            for b in range(NBUF):
                k = g * NBUF + b
                wait_gather(b)
                start_write(k, b)
                bp = (b - 1) % NBUF
                kp_next = k - 1 + NBUF

                @pl.when(k >= 1)
                def _():
                    wait_write(bp)

                    @pl.when(kp_next < NCHUNK)
                    def _():
                        start_gather(kp_next, bp)

        wait_write((NCHUNK - 1) % NBUF)

    return gather_kernel


def kernel(x, table):
    batch, hist = x.shape
    xt = x.astype(jnp.int32).T
    out_t = _make_gather(batch, hist)(xt, table)
    return jnp.transpose(out_t, (1, 0, 2))


# speedup vs baseline: 1.0071x; 1.0071x over previous
import functools

import jax
import jax.numpy as jnp
from jax import lax
from jax.experimental import pallas as pl
from jax.experimental.pallas import tpu as pltpu
from jax.experimental.pallas import tpu_sc as plsc

D = 128
NC, NS = 2, 16
NW = NC * NS
CH = 64
NBUF = 8


@functools.lru_cache(maxsize=None)
def _make_gather(batch: int, hist: int):
    BPT = batch // NW
    NCB = BPT // CH
    NCHUNK = hist * NCB
    assert NCHUNK % NBUF == 0
    mesh = plsc.VectorSubcoreMesh(core_axis_name="c", subcore_axis_name="s")

    @functools.partial(
        pl.kernel,
        out_type=jax.ShapeDtypeStruct((hist, batch, D), jnp.float32),
        mesh=mesh,
        scratch_types=[
            pltpu.VMEM((hist, BPT), jnp.int32),
            pltpu.VMEM((NBUF, CH, D), jnp.float32),
            pltpu.SemaphoreType.DMA((NBUF,)),
            pltpu.SemaphoreType.DMA((NBUF,)),
        ],
    )
    def gather_kernel(idx_hbm, table_hbm, out_hbm, idx_v, rows_v, gsem, wsem):
        wid = lax.axis_index("s") * NC + lax.axis_index("c")
        base = wid * BPT
        pltpu.sync_copy(idx_hbm.at[:, pl.ds(base, BPT)], idx_v)

        def start_gather(k, b):
            h, c = k // NCB, k % NCB
            pltpu.async_copy(
                table_hbm.at[idx_v.at[h].at[pl.ds(c * CH, CH)]],
                rows_v.at[b], gsem.at[b])

        def wait_gather(b):
            pltpu.make_async_copy(table_hbm.at[idx_v.at[0].at[pl.ds(0, CH)]],
                                  rows_v.at[b], gsem.at[b]).wait()

        def start_write(k, b):
            h, c = k // NCB, k % NCB
            pltpu.async_copy(rows_v.at[b],
                             out_hbm.at[h].at[pl.ds(base + c * CH, CH)],
                             wsem.at[b])

        def wait_write(b):
            pltpu.make_async_copy(rows_v.at[b], out_hbm.at[0].at[pl.ds(0, CH)],
                                  wsem.at[b]).wait()

        for b in range(NBUF):
            start_gather(b, b)

        @pl.loop(0, NCHUNK // NBUF)
        def _group(g):
            for b in range(NBUF):
                k = g * NBUF + b
                wait_gather(b)
                start_write(k, b)
                bp = (b - 1) % NBUF
                kp_next = k - 1 + NBUF

                @pl.when(k >= 1)
                def _():
                    wait_write(bp)

                    @pl.when(kp_next < NCHUNK)
                    def _():
                        start_gather(kp_next, bp)

        wait_write((NCHUNK - 1) % NBUF)

    return gather_kernel


def kernel(x, table):
    batch, hist = x.shape
    xt = x.astype(jnp.int32).T
    out_t = _make_gather(batch, hist)(xt, table)
    return jnp.transpose(out_t, (1, 0, 2))


# speedup vs baseline: 1.0106x; 1.0035x over previous
import functools

import jax
import jax.numpy as jnp
from jax import lax
from jax.experimental import pallas as pl
from jax.experimental.pallas import tpu as pltpu
from jax.experimental.pallas import tpu_sc as plsc

D = 128
NC, NS = 2, 16
NW = NC * NS
CH = 64
NBUF = 10


@functools.lru_cache(maxsize=None)
def _make_gather(batch: int, hist: int):
    BPT = batch // NW
    NCB = BPT // CH
    NCHUNK = hist * NCB
    assert NCHUNK % NBUF == 0
    mesh = plsc.VectorSubcoreMesh(core_axis_name="c", subcore_axis_name="s")

    @functools.partial(
        pl.kernel,
        out_type=jax.ShapeDtypeStruct((hist, batch, D), jnp.float32),
        mesh=mesh,
        scratch_types=[
            pltpu.VMEM((hist, BPT), jnp.int32),
            pltpu.VMEM((NBUF, CH, D), jnp.float32),
            pltpu.SemaphoreType.DMA((NBUF,)),
            pltpu.SemaphoreType.DMA((NBUF,)),
        ],
    )
    def gather_kernel(idx_hbm, table_hbm, out_hbm, idx_v, rows_v, gsem, wsem):
        wid = lax.axis_index("s") * NC + lax.axis_index("c")
        base = wid * BPT
        pltpu.sync_copy(idx_hbm.at[:, pl.ds(base, BPT)], idx_v)

        def start_gather(k, b):
            h, c = k // NCB, k % NCB
            pltpu.async_copy(
                table_hbm.at[idx_v.at[h].at[pl.ds(c * CH, CH)]],
                rows_v.at[b], gsem.at[b])

        def wait_gather(b):
            pltpu.make_async_copy(table_hbm.at[idx_v.at[0].at[pl.ds(0, CH)]],
                                  rows_v.at[b], gsem.at[b]).wait()

        def start_write(k, b):
            h, c = k // NCB, k % NCB
            pltpu.async_copy(rows_v.at[b],
                             out_hbm.at[h].at[pl.ds(base + c * CH, CH)],
                             wsem.at[b])

        def wait_write(b):
            pltpu.make_async_copy(rows_v.at[b], out_hbm.at[0].at[pl.ds(0, CH)],
                                  wsem.at[b]).wait()

        for b in range(NBUF):
            start_gather(b, b)

        @pl.loop(0, NCHUNK // NBUF)
        def _group(g):
            for b in range(NBUF):
                k = g * NBUF + b
                wait_gather(b)
                start_write(k, b)
                bp = (b - 1) % NBUF
                kp_next = k - 1 + NBUF

                @pl.when(k >= 1)
                def _():
                    wait_write(bp)

                    @pl.when(kp_next < NCHUNK)
                    def _():
                        start_gather(kp_next, bp)

        wait_write((NCHUNK - 1) % NBUF)

    return gather_kernel


def kernel(x, table):
    batch, hist = x.shape
    xt = x.astype(jnp.int32).T
    out_t = _make_gather(batch, hist)(xt, table)
    return jnp.transpose(out_t, (1, 0, 2))
